# baseline (speedup 1.0000x reference)
import jax
import jax.numpy as jnp
from jax.experimental import pallas as pl
from jax.experimental.pallas import tpu as pltpu

_BLOCK_K = 10240


def _nn_body(q_ref, kt_ref, min_ref, idx_ref):
    i = pl.program_id(0)

    q = q_ref[...]
    kt2 = kt_ref[...]

    qk2 = jax.lax.dot_general(
        q, kt2, (((1,), (0,)), ((), ())),
        preferred_element_type=jnp.float32)
    q_sq = jnp.sum(q * q, axis=1, keepdims=True)
    k_sq = 0.25 * jnp.sum(kt2 * kt2, axis=0, keepdims=True)

    chunk = 128
    nch = _BLOCK_K // chunk
    n_q = q.shape[0]
    rg = 128
    lanef = jax.lax.broadcasted_iota(
        jnp.int32, (1, chunk), 1).astype(jnp.float32)
    basef = (i * _BLOCK_K).astype(jnp.float32)
    mins, ams = [], []
    for g in range(n_q // rg):
        qs = q_sq[g * rg:(g + 1) * rg, :]
        accv = acci = None
        for c in range(nch):
            qkc = qk2[g * rg:(g + 1) * rg, c * chunk:(c + 1) * chunk]
            ksqc = k_sq[:, c * chunk:(c + 1) * chunk]
            dch = (qs + qkc) + ksqc
            idx_row = lanef + (basef + float(c * chunk))
            if c == 0:
                accv = dch
                acci = jnp.broadcast_to(idx_row, dch.shape)
            else:
                mask = dch < accv
                acci = jnp.where(mask, idx_row, acci)
                accv = jnp.minimum(accv, dch)
        mg = jnp.min(accv, axis=1, keepdims=True)
        ag = jnp.min(jnp.where(accv == mg, acci, 3e9),
                     axis=1, keepdims=True)
        mins.append(mg)
        ams.append(ag)
    m = jnp.concatenate(mins, axis=0)
    am = jnp.concatenate(ams, axis=0)

    @pl.when(i == 0)
    def _():
        min_ref[...] = m
        idx_ref[...] = am

    @pl.when(i > 0)
    def _():
        better = m < min_ref[...]
        idx_ref[...] = jnp.where(better, am, idx_ref[...])
        min_ref[...] = jnp.where(better, m, min_ref[...])


def kernel(queries, keys):
    n_q, dim = queries.shape
    n_k = keys.shape[0]
    grid = pl.cdiv(n_k, _BLOCK_K)
    pad = grid * _BLOCK_K - n_k
    kt = jnp.pad(-2.0 * keys.T, ((0, 0), (0, pad)), constant_values=-200.0)
    minv, idxf = pl.pallas_call(
        _nn_body,
        grid=(grid,),
        in_specs=[
            pl.BlockSpec((n_q, dim), lambda i: (0, 0)),
            pl.BlockSpec((dim, _BLOCK_K), lambda i: (0, i)),
        ],
        out_specs=[
            pl.BlockSpec((n_q, 1), lambda i: (0, 0)),
            pl.BlockSpec((n_q, 1), lambda i: (0, 0)),
        ],
        out_shape=[
            jax.ShapeDtypeStruct((n_q, 1), jnp.float32),
            jax.ShapeDtypeStruct((n_q, 1), jnp.float32),
        ],
        compiler_params=pltpu.CompilerParams(
            dimension_semantics=("arbitrary",),
        ),
    )(queries, kt)
    scores = jnp.sqrt(jnp.maximum(minv[:, 0], 0.0) + 1e-12)
    idx = idxf.astype(jnp.int32)
    return (scores, idx)


# speedup vs baseline: 1.0424x; 1.0424x over previous
import functools

import jax
import jax.numpy as jnp
from jax.experimental import pallas as pl
from jax.experimental.pallas import tpu as pltpu

_BLOCK_K = 10240


def _nn_body(n_k, q_ref, kt_ref, min_ref, idx_ref):
    i = pl.program_id(0)

    q = q_ref[...]
    kt2 = kt_ref[...]
    col16 = jax.lax.broadcasted_iota(jnp.int32, kt2.shape, 1) + i * _BLOCK_K
    kt2 = jnp.where(col16 < n_k, kt2, -200.0)

    qk2 = jax.lax.dot_general(
        q, kt2, (((1,), (0,)), ((), ())),
        preferred_element_type=jnp.float32)
    q_sq = jnp.sum(q * q, axis=1, keepdims=True)
    k_sq = 0.25 * jnp.sum(kt2 * kt2, axis=0, keepdims=True)

    chunk = 128
    nch = _BLOCK_K // chunk
    n_q = q.shape[0]
    rg = 128
    lanef = jax.lax.broadcasted_iota(
        jnp.int32, (1, chunk), 1).astype(jnp.float32)
    basef = (i * _BLOCK_K).astype(jnp.float32)
    mins, ams = [], []
    for g in range(n_q // rg):
        qs = q_sq[g * rg:(g + 1) * rg, :]
        accv = acci = None
        for c in range(nch):
            qkc = qk2[g * rg:(g + 1) * rg, c * chunk:(c + 1) * chunk]
            ksqc = k_sq[:, c * chunk:(c + 1) * chunk]
            dch = (qs + qkc) + ksqc
            idx_row = lanef + (basef + float(c * chunk))
            if c == 0:
                accv = dch
                acci = jnp.broadcast_to(idx_row, dch.shape)
            else:
                mask = dch < accv
                acci = jnp.where(mask, idx_row, acci)
                accv = jnp.minimum(accv, dch)
        mg = jnp.min(accv, axis=1, keepdims=True)
        ag = jnp.min(jnp.where(accv == mg, acci, 3e9),
                     axis=1, keepdims=True)
        mins.append(mg)
        ams.append(ag)
    m = jnp.concatenate(mins, axis=0)
    am = jnp.concatenate(ams, axis=0)

    @pl.when(i == 0)
    def _():
        min_ref[...] = m
        idx_ref[...] = am

    @pl.when(i > 0)
    def _():
        better = m < min_ref[...]
        idx_ref[...] = jnp.where(better, am, idx_ref[...])
        min_ref[...] = jnp.where(better, m, min_ref[...])


def kernel(queries, keys):
    n_q, dim = queries.shape
    n_k = keys.shape[0]
    kt = -2.0 * keys.T

    grid = pl.cdiv(n_k, _BLOCK_K)
    minv, idxf = pl.pallas_call(
        functools.partial(_nn_body, n_k),
        grid=(grid,),
        in_specs=[
            pl.BlockSpec((n_q, dim), lambda i: (0, 0)),
            pl.BlockSpec((dim, _BLOCK_K), lambda i: (0, i)),
        ],
        out_specs=[
            pl.BlockSpec((n_q, 1), lambda i: (0, 0)),
            pl.BlockSpec((n_q, 1), lambda i: (0, 0)),
        ],
        out_shape=[
            jax.ShapeDtypeStruct((n_q, 1), jnp.float32),
            jax.ShapeDtypeStruct((n_q, 1), jnp.float32),
        ],
        compiler_params=pltpu.CompilerParams(
            dimension_semantics=("arbitrary",),
        ),
    )(queries, kt)
    scores = jnp.sqrt(jnp.maximum(minv[:, 0], 0.0) + 1e-12)
    idx = idxf.astype(jnp.int32)
    return (scores, idx)
